# Initial kernel scaffold; baseline (speedup 1.0000x reference)
import jax
import jax.numpy as jnp
from jax.experimental import pallas as pl


def kernel(student_idx, course_idx, term_idx, college_idx, major_idx, hist_cont, course_cont, W_student, W_course, W_term, W_college, W_major, W_hist, b_hist, W_cc, b_cc):
    raise NotImplementedError("write your pallas kernel here")


# speedup vs baseline: 8.6870x; 8.6870x over previous
import functools

import jax
import jax.numpy as jnp
from jax import lax
from jax.experimental import pallas as pl
from jax.experimental.pallas import tpu as pltpu
from jax.experimental.pallas import tpu_sc as plsc

_B = 16384
_L = 50
_D_ID = 64
_NC = 2
_NS = 16
_NW = _NC * _NS
_ROWS_W = _B // _NW
_CHUNK_B = 2
_CHUNK_I = _CHUNK_B * _L
_NCHUNK = _ROWS_W // _CHUNK_B
_LANE = 16


def _sc_gather(course_idx2d, student_idx2d, w_course, w_student):
    mesh = plsc.VectorSubcoreMesh(core_axis_name="c", subcore_axis_name="s")

    @functools.partial(
        pl.kernel,
        out_type=[
            jax.ShapeDtypeStruct((_B, _D_ID), jnp.float32),
            jax.ShapeDtypeStruct((_B, _D_ID), jnp.float32),
        ],
        mesh=mesh,
        scratch_types=[
            pltpu.VMEM((_NCHUNK, _CHUNK_I), jnp.int32),
            pltpu.VMEM((_CHUNK_I, _D_ID), jnp.float32),
            pltpu.VMEM((_CHUNK_I, _D_ID), jnp.float32),
            pltpu.VMEM((_ROWS_W, _D_ID), jnp.float32),
            pltpu.VMEM((4, 128), jnp.int32),
            pltpu.VMEM((128, _D_ID), jnp.float32),
            pltpu.SemaphoreType.DMA,
            pltpu.SemaphoreType.DMA,
            pltpu.SemaphoreType.DMA,
        ],
        compiler_params=pltpu.CompilerParams(use_tc_tiling_on_sc=False),
    )
    def k(cidx_hbm, sidx_hbm, wc_hbm, ws_hbm, crs_out, stu_out,
          cidx_v, buf_a, buf_b, out_v, sidx_v, srows_v, sem_a, sem_b, sem_s):
        wid = lax.axis_index("s") * _NC + lax.axis_index("c")

        pltpu.sync_copy(cidx_hbm.at[pl.ds(wid * _NCHUNK, _NCHUNK)], cidx_v)

        def issue(c, buf, sem):
            pltpu.async_copy(wc_hbm.at[cidx_v.at[c]], buf, sem)

        def wait(c, buf, sem):
            pltpu.make_async_copy(wc_hbm.at[cidx_v.at[c]], buf, sem).wait()

        def reduce_chunk(c, buf):
            for r in range(_CHUNK_B):
                def body(l, accs, _r=r):
                    return tuple(
                        a + buf[_r * _L + l, pl.ds(g * _LANE, _LANE)]
                        for g, a in enumerate(accs)
                    )
                accs = lax.fori_loop(
                    0, _L, body,
                    tuple(jnp.zeros((_LANE,), jnp.float32)
                          for _ in range(_D_ID // _LANE)))
                row = c * _CHUNK_B + r
                for g in range(_D_ID // _LANE):
                    out_v[row, pl.ds(g * _LANE, _LANE)] = accs[g] * (1.0 / _L)

        issue(0, buf_a, sem_a)
        issue(1, buf_b, sem_b)

        @pl.loop(0, _NCHUNK, step=2)
        def _(c):
            wait(c, buf_a, sem_a)
            reduce_chunk(c, buf_a)

            @pl.when(c + 2 < _NCHUNK)
            def _():
                issue(c + 2, buf_a, sem_a)

            wait(c + 1, buf_b, sem_b)
            reduce_chunk(c + 1, buf_b)

            @pl.when(c + 3 < _NCHUNK)
            def _():
                issue(c + 3, buf_b, sem_b)

        pltpu.sync_copy(out_v, crs_out.at[pl.ds(wid * _ROWS_W, _ROWS_W)])

        pltpu.sync_copy(sidx_hbm.at[pl.ds(wid * 4, 4)], sidx_v)
        for j in range(4):
            pltpu.async_copy(ws_hbm.at[sidx_v.at[j]], srows_v, sem_s).wait()
            pltpu.sync_copy(
                srows_v, stu_out.at[pl.ds(wid * _ROWS_W + j * 128, 128)])

    return k(course_idx2d, student_idx2d, w_course, w_student)


_BLK = 512


def _tc_body(hist_ref, term_ref, col_ref, maj_ref, cc_ref,
             wh_ref, bh_ref, wt_ref, wcol_ref, wmaj_ref, wcc_ref, bcc_ref,
             out_ref):
    hist = hist_ref[...]
    hproj = (jnp.dot(hist, wh_ref[...], preferred_element_type=jnp.float32)
             * (1.0 / _L) + bh_ref[...])

    term = term_ref[...]
    bins = lax.broadcasted_iota(jnp.int32, (1, 64), 1)
    counts = jnp.zeros((_BLK, 64), jnp.float32)
    for l in range(_L):
        counts = counts + (term[:, l:l + 1] == bins).astype(jnp.float32)
    term_mean = jnp.dot(counts, wt_ref[...],
                        preferred_element_type=jnp.float32) * (1.0 / _L)

    col_oh = (col_ref[...] == lax.broadcasted_iota(jnp.int32, (1, 32), 1)
              ).astype(jnp.float32)
    e_col = jnp.dot(col_oh, wcol_ref[...], preferred_element_type=jnp.float32)

    maj_oh = (maj_ref[...] == lax.broadcasted_iota(jnp.int32, (1, 256), 1)
              ).astype(jnp.float32)
    e_maj = jnp.dot(maj_oh, wmaj_ref[...], preferred_element_type=jnp.float32)

    cc = cc_ref[...]
    wcc = wcc_ref[...]
    c_proj = cc[:, 0:1] * wcc[0:1, :] + cc[:, 1:2] * wcc[1:2, :] + bcc_ref[...]

    out_ref[...] = jnp.concatenate(
        [term_mean, hproj, e_col, e_maj, c_proj], axis=1)


def _tc_dense(hist_flat, term_idx, col2, maj2, course_cont,
              wh_rep, bh2, wt_pad, wcol_pad, wmaj_pad, w_cc, bcc2):
    grid = (_B // _BLK,)
    full = lambda shape: pl.BlockSpec(shape, lambda i: (0, 0))
    blk = lambda minor: pl.BlockSpec((_BLK, minor), lambda i: (i, 0))
    return pl.pallas_call(
        _tc_body,
        grid=grid,
        in_specs=[
            blk(_L * 16),
            blk(_L),
            blk(1),
            blk(1),
            blk(2),
            full((_L * 16, 16)),
            full((1, 16)),
            full((64, 32)),
            full((32, 16)),
            full((256, 16)),
            full((2, 16)),
            full((1, 16)),
        ],
        out_specs=blk(96),
        out_shape=jax.ShapeDtypeStruct((_B, 96), jnp.float32),
    )(hist_flat, term_idx, col2, maj2, course_cont,
      wh_rep, bh2, wt_pad, wcol_pad, wmaj_pad, w_cc, bcc2)


def kernel(student_idx, course_idx, term_idx, college_idx, major_idx,
           hist_cont, course_cont,
           W_student, W_course, W_term, W_college, W_major,
           W_hist, b_hist, W_cc, b_cc):
    cidx2 = course_idx.astype(jnp.int32).reshape(_B * _L // _CHUNK_I, _CHUNK_I)
    sidx2 = student_idx.astype(jnp.int32).reshape(128, 128)
    crs_mean, stu = _sc_gather(cidx2, sidx2, W_course, W_student)

    hist_flat = hist_cont.reshape(_B, _L * 16)
    wh_rep = jnp.tile(W_hist, (_L, 1))
    wt_pad = jnp.zeros((64, 32), jnp.float32).at[:51].set(W_term)
    wcol_pad = jnp.zeros((32, 16), jnp.float32).at[:31].set(W_college)
    wmaj_pad = jnp.zeros((256, 16), jnp.float32).at[:201].set(W_major)
    tc = _tc_dense(hist_flat, term_idx.astype(jnp.int32),
                   college_idx.astype(jnp.int32).reshape(_B, 1),
                   major_idx.astype(jnp.int32).reshape(_B, 1),
                   course_cont, wh_rep, b_hist.reshape(1, 16),
                   wt_pad, wcol_pad, wmaj_pad, W_cc, b_cc.reshape(1, 16))

    return jnp.concatenate([crs_mean, tc[:, :48], stu, tc[:, 48:]], axis=1)


# speedup vs baseline: 8.6994x; 1.0014x over previous
import functools

import jax
import jax.numpy as jnp
from jax import lax
from jax.experimental import pallas as pl
from jax.experimental.pallas import tpu as pltpu
from jax.experimental.pallas import tpu_sc as plsc

_B = 16384
_L = 50
_D_ID = 64
_N_STU = 1000000
_NC = 2
_NS = 16
_NW = _NC * _NS
_ROWS_W = _B // _NW
_CHUNK_B = 2
_CHUNK_I = _CHUNK_B * _L
_NCHUNK = _ROWS_W // _CHUNK_B
_LANE = 16


def _sc_gather(course_idx2d, student_pair_idx, student_par, w_course, ws_pairs):
    mesh = plsc.VectorSubcoreMesh(core_axis_name="c", subcore_axis_name="s")

    @functools.partial(
        pl.kernel,
        out_type=[
            jax.ShapeDtypeStruct((_B, _D_ID), jnp.float32),
            jax.ShapeDtypeStruct((_B, _D_ID), jnp.float32),
        ],
        mesh=mesh,
        scratch_types=[
            pltpu.VMEM((_NCHUNK, _CHUNK_I), jnp.int32),
            pltpu.VMEM((_CHUNK_I, _D_ID), jnp.float32),
            pltpu.VMEM((_CHUNK_I, _D_ID), jnp.float32),
            pltpu.VMEM((_ROWS_W, _D_ID), jnp.float32),
            pltpu.VMEM((4, 128), jnp.int32),
            pltpu.VMEM((4, 128), jnp.int32),
            pltpu.VMEM((128, 128), jnp.float32),
            pltpu.VMEM((128, _D_ID), jnp.float32),
            pltpu.SemaphoreType.DMA,
            pltpu.SemaphoreType.DMA,
            pltpu.SemaphoreType.DMA,
        ],
        compiler_params=pltpu.CompilerParams(use_tc_tiling_on_sc=False,
                                             needs_layout_passes=False),
    )
    def k(cidx_hbm, sidx_hbm, spar_hbm, wc_hbm, wsp_hbm, crs_out, stu_out,
          cidx_v, buf_a, buf_b, out_v, sidx_v, spar_v, pairs_v, sx_v,
          sem_a, sem_b, sem_s):
        wid = lax.axis_index("s") * _NC + lax.axis_index("c")

        pltpu.sync_copy(cidx_hbm.at[pl.ds(wid * _NCHUNK, _NCHUNK)], cidx_v)

        def issue(c, buf, sem):
            pltpu.async_copy(wc_hbm.at[cidx_v.at[c]], buf, sem)

        def wait(c, buf, sem):
            pltpu.make_async_copy(wc_hbm.at[cidx_v.at[c]], buf, sem).wait()

        def reduce_chunk(c, buf):
            for r in range(_CHUNK_B):
                def body(l, accs, _r=r):
                    return tuple(
                        a + buf[_r * _L + l, pl.ds(g * _LANE, _LANE)]
                        for g, a in enumerate(accs)
                    )
                accs = lax.fori_loop(
                    0, _L, body,
                    tuple(jnp.zeros((_LANE,), jnp.float32)
                          for _ in range(_D_ID // _LANE)))
                row = c * _CHUNK_B + r
                for g in range(_D_ID // _LANE):
                    out_v[row, pl.ds(g * _LANE, _LANE)] = accs[g] * (1.0 / _L)

        issue(0, buf_a, sem_a)
        issue(1, buf_b, sem_b)

        @pl.loop(0, _NCHUNK, step=2)
        def _(c):
            wait(c, buf_a, sem_a)
            reduce_chunk(c, buf_a)

            @pl.when(c + 2 < _NCHUNK)
            def _():
                issue(c + 2, buf_a, sem_a)

            wait(c + 1, buf_b, sem_b)
            reduce_chunk(c + 1, buf_b)

            @pl.when(c + 3 < _NCHUNK)
            def _():
                issue(c + 3, buf_b, sem_b)

        pltpu.sync_copy(out_v, crs_out.at[pl.ds(wid * _ROWS_W, _ROWS_W)])

        pltpu.sync_copy(sidx_hbm.at[pl.ds(wid * 4, 4)], sidx_v)
        pltpu.sync_copy(spar_hbm.at[pl.ds(wid * 4, 4)], spar_v)
        iota16 = lax.iota(jnp.int32, 16)
        for j in range(4):
            pltpu.async_copy(wsp_hbm.at[sidx_v.at[j]], pairs_v, sem_s).wait()

            @pl.loop(0, 128)
            def _(r, _j=j):
                row_splat = jnp.full((16,), 0, jnp.int32) + r
                par = plsc.load_gather(
                    spar_v, [jnp.full((16,), _j, jnp.int32), row_splat])
                colbase = par * 64
                for g in range(_D_ID // _LANE):
                    sx_v[r, pl.ds(g * _LANE, _LANE)] = plsc.load_gather(
                        pairs_v, [row_splat, colbase + (iota16 + g * _LANE)])

            pltpu.sync_copy(
                sx_v, stu_out.at[pl.ds(wid * _ROWS_W + j * 128, 128)])

    return k(course_idx2d, student_pair_idx, student_par, w_course, ws_pairs)


_BLK = 512


def _tc_body(hist_ref, term_ref, col_ref, maj_ref, cc_ref,
             wh_ref, bh_ref, wt_ref, wcol_ref, wmaj_ref, wcc_ref, bcc_ref,
             out_ref):
    hist = hist_ref[...]
    hproj = (jnp.dot(hist, wh_ref[...], preferred_element_type=jnp.float32)
             * (1.0 / _L) + bh_ref[...])

    term = term_ref[...]
    bins = lax.broadcasted_iota(jnp.int32, (1, 64), 1)
    counts = jnp.zeros((_BLK, 64), jnp.float32)
    for l in range(_L):
        counts = counts + (term[:, l:l + 1] == bins).astype(jnp.float32)
    term_mean = jnp.dot(counts, wt_ref[...],
                        preferred_element_type=jnp.float32) * (1.0 / _L)

    col_oh = (col_ref[...] == lax.broadcasted_iota(jnp.int32, (1, 32), 1)
              ).astype(jnp.float32)
    e_col = jnp.dot(col_oh, wcol_ref[...], preferred_element_type=jnp.float32)

    maj_oh = (maj_ref[...] == lax.broadcasted_iota(jnp.int32, (1, 256), 1)
              ).astype(jnp.float32)
    e_maj = jnp.dot(maj_oh, wmaj_ref[...], preferred_element_type=jnp.float32)

    cc = cc_ref[...]
    wcc = wcc_ref[...]
    c_proj = cc[:, 0:1] * wcc[0:1, :] + cc[:, 1:2] * wcc[1:2, :] + bcc_ref[...]

    out_ref[...] = jnp.concatenate(
        [term_mean, hproj, e_col, e_maj, c_proj], axis=1)


def _tc_dense(hist_flat, term_idx, col2, maj2, course_cont,
              wh_rep, bh2, wt_pad, wcol_pad, wmaj_pad, w_cc, bcc2):
    grid = (_B // _BLK,)
    full = lambda shape: pl.BlockSpec(shape, lambda i: (0, 0))
    blk = lambda minor: pl.BlockSpec((_BLK, minor), lambda i: (i, 0))
    return pl.pallas_call(
        _tc_body,
        grid=grid,
        in_specs=[
            blk(_L * 16),
            blk(_L),
            blk(1),
            blk(1),
            blk(2),
            full((_L * 16, 16)),
            full((1, 16)),
            full((64, 32)),
            full((32, 16)),
            full((256, 16)),
            full((2, 16)),
            full((1, 16)),
        ],
        out_specs=blk(96),
        out_shape=jax.ShapeDtypeStruct((_B, 96), jnp.float32),
    )(hist_flat, term_idx, col2, maj2, course_cont,
      wh_rep, bh2, wt_pad, wcol_pad, wmaj_pad, w_cc, bcc2)


def kernel(student_idx, course_idx, term_idx, college_idx, major_idx,
           hist_cont, course_cont,
           W_student, W_course, W_term, W_college, W_major,
           W_hist, b_hist, W_cc, b_cc):
    cidx2 = course_idx.astype(jnp.int32).reshape(_B * _L // _CHUNK_I, _CHUNK_I)
    sidx = student_idx.astype(jnp.int32)
    ws_pairs = W_student[:_N_STU].reshape(_N_STU // 2, 2 * _D_ID)
    crs_mean, stu = _sc_gather(cidx2,
                               (sidx >> 1).reshape(128, 128),
                               (sidx & 1).reshape(128, 128),
                               W_course, ws_pairs)

    hist_flat = hist_cont.reshape(_B, _L * 16)
    wh_rep = jnp.tile(W_hist, (_L, 1))
    wt_pad = jnp.zeros((64, 32), jnp.float32).at[:51].set(W_term)
    wcol_pad = jnp.zeros((32, 16), jnp.float32).at[:31].set(W_college)
    wmaj_pad = jnp.zeros((256, 16), jnp.float32).at[:201].set(W_major)
    tc = _tc_dense(hist_flat, term_idx.astype(jnp.int32),
                   college_idx.astype(jnp.int32).reshape(_B, 1),
                   major_idx.astype(jnp.int32).reshape(_B, 1),
                   course_cont, wh_rep, b_hist.reshape(1, 16),
                   wt_pad, wcol_pad, wmaj_pad, W_cc, b_cc.reshape(1, 16))

    return jnp.concatenate([crs_mean, tc[:, :48], stu, tc[:, 48:]], axis=1)
